# Initial kernel scaffold; baseline (speedup 1.0000x reference)
import jax
import jax.numpy as jnp
from jax.experimental import pallas as pl


def kernel(x, edge_index, w_low0, w_high0, w_mlp0, avl0, avh0, avm0, attv0, w_low1, w_high1, w_mlp1, avl1, avh1, avm1, attv1):
    raise NotImplementedError("write your pallas kernel here")


# speedup vs baseline: 3.3343x; 3.3343x over previous
import functools

import jax
import jax.numpy as jnp
from jax import lax
from jax.experimental import pallas as pl
from jax.experimental.pallas import tpu as pltpu
from jax.experimental.pallas import tpu_sc as plsc

N = 10000
E = 160000
C = 256
H = 128
K = 80
EPT = E // 16
NCHUNK = EPT // K
RO = 40
NRC = N // RO
NRO = -(-NRC // 16)
LB = 16
_DBG_SC_ACC = True
_DBG_SC_DEG = False
_DBG_GATHER = True


def _spmm_body(with_deg, refs):
    if with_deg:
        (xs, rowi, coli, zs, dego,
         colb, rowb, rowsv, onesv, acc, degs, obuf, xbuf, dbuf, sem) = refs
    else:
        (xs, rowi, coli, degi, zs,
         colb, rowb, rowsv, onesv, acc, degs, obuf, xbuf, dbuf, sem) = refs

    c = lax.axis_index("c")
    s = lax.axis_index("s")
    feat_off = c * N

    def fill_obuf(r, _):
        def fcol(j, _):
            obuf[r, pl.ds(j * LB, LB)] = jnp.zeros((LB,), jnp.float32)
            return 0
        return lax.fori_loop(0, H // LB, fcol, 0)
    lax.fori_loop(0, RO, fill_obuf, 0)

    def fill_dbuf(r, _):
        dbuf[r, :] = jnp.zeros((LB,), jnp.float32)
        return 0
    lax.fori_loop(0, RO, fill_dbuf, 0)

    if with_deg:
        def fill_ones(r, _):
            onesv[r, :] = jnp.ones((LB,), jnp.float32)
            return 0
        lax.fori_loop(0, K, fill_ones, 0)

    def zero_chunk(k, _):
        idx = k * 16 + s
        @pl.when(idx < NRC)
        def _():
            r0 = idx * RO
            pltpu.sync_copy(obuf, acc.at[pl.ds(r0, RO)])
            if with_deg:
                pltpu.sync_copy(dbuf, degs.at[pl.ds(r0, RO)])
        return 0
    lax.fori_loop(0, NRO, zero_chunk, 0)

    plsc.subcore_barrier()

    def edge_chunk(j, _):
        e0 = s * EPT + j * K
        pltpu.sync_copy(coli.at[pl.ds(e0, K)], colb.at[0, 0])
        pltpu.sync_copy(rowi.at[pl.ds(e0, K)], rowb.at[0, 0])
        if with_deg and _DBG_SC_DEG:
            pltpu.sync_copy(onesv, degs.at[colb.at[0, 0]], add=True)
        def shift(t, _):
            colb[0, 0, pl.ds(t * LB, LB)] = colb[0, 0, pl.ds(t * LB, LB)] + feat_off
            return 0
        lax.fori_loop(0, K // LB, shift, 0)
        if _DBG_GATHER:
            pltpu.async_copy(xs.at[colb.at[0, 0]], rowsv, sem).wait()
        if _DBG_SC_ACC:
            def sgrp(g, _):
                idxv = rowb[0, 0, pl.ds(g * LB, LB)]
                pltpu.async_copy(rowsv.at[pl.ds(g * LB, LB)], acc.at[idxv],
                                 sem, add=True).wait()
                return 0
            lax.fori_loop(0, K // LB, sgrp, 0)
        return 0
    lax.fori_loop(0, NCHUNK, edge_chunk, 0)

    plsc.subcore_barrier()

    def out_chunk(k, _):
        idx = k * 16 + s
        @pl.when(idx < NRC)
        def _():
            _out_one_chunk(with_deg, idx, feat_off, c, refs)
        return 0
    lax.fori_loop(0, NRO, out_chunk, 0)


def _out_one_chunk(with_deg, idx, feat_off, c, refs):
    if with_deg:
        (xs, rowi, coli, zs, dego,
         colb, rowb, rowsv, onesv, acc, degs, obuf, xbuf, dbuf, sem) = refs
    else:
        (xs, rowi, coli, degi, zs,
         colb, rowb, rowsv, onesv, acc, degs, obuf, xbuf, dbuf, sem) = refs
    if True:
        r0 = idx * RO
        pltpu.sync_copy(acc.at[pl.ds(r0, RO)], obuf)
        pltpu.sync_copy(xs.at[pl.ds(feat_off + r0, RO)], xbuf)
        if with_deg:
            pltpu.sync_copy(degs.at[pl.ds(r0, RO)], dbuf)
        else:
            pltpu.sync_copy(degi.at[pl.ds(r0, RO)], dbuf)

        def orow(r, _):
            dv = dbuf[r, :]
            dinvv = 1.0 / (dv + 1.0)
            dinv = dinvv[0]
            def ocol(j, _):
                v = obuf[r, pl.ds(j * LB, LB)]
                xv = xbuf[r, pl.ds(j * LB, LB)]
                obuf[r, pl.ds(j * LB, LB)] = (v + xv) * dinv
                return 0
            return lax.fori_loop(0, H // LB, ocol, 0)
        lax.fori_loop(0, RO, orow, 0)

        pltpu.sync_copy(obuf, zs.at[pl.ds(feat_off + r0, RO)])
        if with_deg:
            @pl.when(c == 0)
            def _():
                pltpu.sync_copy(dbuf, dego.at[pl.ds(r0, RO)])


def _make_spmm(with_deg):
    mesh = plsc.VectorSubcoreMesh(core_axis_name="c", subcore_axis_name="s")
    zs_t = jax.ShapeDtypeStruct((2 * N, H), jnp.float32)
    deg_t = jax.ShapeDtypeStruct((N, LB), jnp.float32)
    out_type = (zs_t, deg_t) if with_deg else zs_t
    scratch = [
        pltpu.VMEM((1, 1, K), jnp.int32),
        pltpu.VMEM((1, 1, K), jnp.int32),
        pltpu.VMEM((K, H), jnp.float32),
        pltpu.VMEM((K, LB), jnp.float32),
        pltpu.VMEM_SHARED((N, H), jnp.float32),
        pltpu.VMEM_SHARED((N, LB), jnp.float32),
        pltpu.VMEM((RO, H), jnp.float32),
        pltpu.VMEM((RO, H), jnp.float32),
        pltpu.VMEM((RO, LB), jnp.float32),
        pltpu.SemaphoreType.DMA,
    ]

    @functools.partial(pl.kernel, mesh=mesh, out_type=out_type,
                       scratch_types=scratch)
    def run(*refs):
        _spmm_body(with_deg, refs)

    return run


_spmm_deg = _make_spmm(True)
_spmm_nodeg = _make_spmm(False)


BM = 400


def _tc_layer_body(relu_out, x0, x1, z0, z1, wl, wh, wm, avl, avh, avm, attv,
                   o0, o1):
    x = jnp.concatenate([x0[...], x1[...]], axis=1)
    z = jnp.concatenate([z0[...], z1[...]], axis=1)
    f32 = jnp.float32
    low = jnp.maximum(jnp.dot(z, wl[...], preferred_element_type=f32), 0.0)
    high = jnp.maximum(jnp.dot(x - z, wh[...], preferred_element_type=f32), 0.0)
    mlp = jnp.maximum(jnp.dot(x, wm[...], preferred_element_type=f32), 0.0)
    l0 = jnp.sum(low * avl[...], axis=1, keepdims=True)
    l1 = jnp.sum(high * avh[...], axis=1, keepdims=True)
    l2 = jnp.sum(mlp * avm[...], axis=1, keepdims=True)
    sig = jnp.concatenate(
        [1.0 / (1.0 + jnp.exp(-l0)),
         1.0 / (1.0 + jnp.exp(-l1)),
         1.0 / (1.0 + jnp.exp(-l2))], axis=1)
    a = attv[...]
    t = (sig[:, 0:1] * a[0:1, :] + sig[:, 1:2] * a[1:2, :]
         + sig[:, 2:3] * a[2:3, :]) / 3.0
    m = jnp.max(t, axis=1, keepdims=True)
    e = jnp.exp(t - m)
    att = e / jnp.sum(e, axis=1, keepdims=True)
    out = 3.0 * (att[:, 0:1] * low + att[:, 1:2] * high + att[:, 2:3] * mlp)
    if relu_out:
        out = jnp.maximum(out, 0.0)
    o0[...] = out[:, :H]
    o1[...] = out[:, H:]


def _make_tc_layer(relu_out):
    half = pl.BlockSpec((BM, H), lambda i: (i, 0))
    wspec = pl.BlockSpec((C, C), lambda i: (0, 0))
    avspec = pl.BlockSpec((1, C), lambda i: (0, 0))
    attspec = pl.BlockSpec((3, 3), lambda i: (0, 0))
    return pl.pallas_call(
        functools.partial(_tc_layer_body, relu_out),
        grid=(N // BM,),
        in_specs=[half, half, half, half, wspec, wspec, wspec,
                  avspec, avspec, avspec, attspec],
        out_specs=[half, half],
        out_shape=[jax.ShapeDtypeStruct((N, H), jnp.float32),
                   jax.ShapeDtypeStruct((N, H), jnp.float32)],
        compiler_params=pltpu.CompilerParams(
            dimension_semantics=("parallel",)),
    )


_tc_layer1 = _make_tc_layer(True)
_tc_layer2 = _make_tc_layer(False)


def kernel(x, edge_index, w_low0, w_high0, w_mlp0, avl0, avh0, avm0, attv0,
           w_low1, w_high1, w_mlp1, avl1, avh1, avm1, attv1):
    f32 = jnp.float32
    x = x.astype(f32)
    row = edge_index[0].astype(jnp.int32)
    col = edge_index[1].astype(jnp.int32)
    deg = jax.ops.segment_sum(jnp.ones((E,), f32), col, num_segments=N) + 1.0
    dinv = 1.0 / deg

    def agg(y):
        z = jax.ops.segment_sum(y[col], row, num_segments=N)
        return dinv[:, None] * (z + y)

    def layer(y, wl, wh, wm, avl, avh, avm, attv):
        z = agg(y)
        low = jax.nn.relu(z @ wl)
        high = jax.nn.relu((y - z) @ wh)
        mlp = jax.nn.relu(y @ wm)
        logits = jnp.concatenate([low @ avl, high @ avh, mlp @ avm], axis=1)
        att = jax.nn.softmax(jax.nn.sigmoid(logits) @ attv / 3.0, axis=1)
        return 3.0 * (att[:, 0:1] * low + att[:, 1:2] * high + att[:, 2:3] * mlp)

    fea = jax.nn.relu(layer(x, w_low0, w_high0, w_mlp0, avl0, avh0, avm0, attv0))
    return layer(fea, w_low1, w_high1, w_mlp1, avl1, avh1, avm1, attv1)


# speedup vs baseline: 4.9028x; 1.4704x over previous
import functools

import jax
import jax.numpy as jnp
from jax import lax
from jax.experimental import pallas as pl
from jax.experimental.pallas import tpu as pltpu
from jax.experimental.pallas import tpu_sc as plsc

N = 10000
E = 160000
C = 256
LB = 16
F = 4
NPASS = 2
EC = 2000
NEC = E // EC
EPD = E // 16
NECD = EPD // EC
NP = 10240
RNG = NP // 16


def _zero_1d(ref, nwords, base=0):
    def z(i, _):
        ref[pl.ds(base + i * LB, LB)] = jnp.zeros((LB,), jnp.float32)
        return 0
    lax.fori_loop(0, nwords // LB, z, 0)


def _spmm_body(with_deg, refs):
    if with_deg:
        (xT, rowi, coli, zT, dinv_h,
         xb, accb, colst, rowst, degb, dinvb, tmpb, dpart, dsp) = refs
    else:
        (xT, rowi, coli, dinv_i, zT,
         xb, accb, colst, rowst, degb, dinvb, tmpb, dpart, dsp) = refs

    c = lax.axis_index("c")
    s = lax.axis_index("s")
    tid = c * 16 + s
    ones = jnp.ones((LB,), jnp.float32)

    if with_deg:
        _zero_1d(degb, NP)

        def dchunk(j, _):
            e0 = s * EPD + j * EC
            pltpu.sync_copy(coli.at[pl.ds(e0, EC)], colst)

            def dgrp(g, _):
                colv = colst[pl.ds(g * LB, LB)]
                plsc.addupdate_scatter(degb, [colv], ones)
                return 0
            return lax.fori_loop(0, EC // LB, dgrp, 0)
        lax.fori_loop(0, NECD, dchunk, 0)

        pltpu.sync_copy(degb, dpart.at[s])
        plsc.subcore_barrier()

        r0 = s * RNG
        _zero_1d(degb, RNG, base=r0)
        for t2 in range(16):
            pltpu.sync_copy(dpart.at[t2, pl.ds(r0, RNG)], tmpb)

            def radd(i, _):
                o = r0 + i * LB
                degb[pl.ds(o, LB)] = degb[pl.ds(o, LB)] + tmpb[pl.ds(i * LB, LB)]
                return 0
            lax.fori_loop(0, RNG // LB, radd, 0)

        def rinv(i, _):
            o = r0 + i * LB
            degb[pl.ds(o, LB)] = 1.0 / (degb[pl.ds(o, LB)] + 1.0)
            return 0
        lax.fori_loop(0, RNG // LB, rinv, 0)

        pltpu.sync_copy(degb.at[pl.ds(r0, RNG)], dsp.at[pl.ds(r0, RNG)])

        @pl.when(c == 0)
        def _():
            pltpu.sync_copy(degb.at[pl.ds(r0, RNG)], dinv_h.at[pl.ds(r0, RNG)])
        plsc.subcore_barrier()
        pltpu.sync_copy(dsp, dinvb)
    else:
        pltpu.sync_copy(dinv_i, dinvb)

    for p in range(NPASS):
        fb = (tid * F + p * F * 32) * N
        pltpu.sync_copy(xT.at[pl.ds(fb, F * N)], xb)
        _zero_1d(accb, F * N)

        def echunk(j, _):
            e0 = j * EC
            pltpu.sync_copy(coli.at[pl.ds(e0, EC)], colst)
            pltpu.sync_copy(rowi.at[pl.ds(e0, EC)], rowst)

            def egrp(g, _):
                colv = colst[pl.ds(g * LB, LB)]
                rowv = rowst[pl.ds(g * LB, LB)]
                for f in range(F):
                    vals = plsc.load_gather(xb, [colv + (f * N)])
                    plsc.addupdate_scatter(accb, [rowv + (f * N)], vals)
                return 0
            return lax.fori_loop(0, EC // LB, egrp, 0)
        lax.fori_loop(0, NEC, echunk, 0)

        for f in range(F):
            def fin(i, _):
                o = f * N + i * LB
                a = accb[pl.ds(o, LB)]
                xv = xb[pl.ds(o, LB)]
                dv = dinvb[pl.ds(i * LB, LB)]
                accb[pl.ds(o, LB)] = (a + xv) * dv
                return 0
            lax.fori_loop(0, N // LB, fin, 0)
            pltpu.sync_copy(accb.at[pl.ds(f * N, N)],
                            zT.at[pl.ds(fb + f * N, N)])


def _make_spmm(with_deg):
    mesh = plsc.VectorSubcoreMesh(core_axis_name="c", subcore_axis_name="s")
    zT_t = jax.ShapeDtypeStruct((C * N,), jnp.float32)
    dinv_t = jax.ShapeDtypeStruct((NP,), jnp.float32)
    out_type = (zT_t, dinv_t) if with_deg else zT_t
    scratch = [
        pltpu.VMEM((F * N,), jnp.float32),
        pltpu.VMEM((F * N,), jnp.float32),
        pltpu.VMEM((EC,), jnp.int32),
        pltpu.VMEM((EC,), jnp.int32),
        pltpu.VMEM((NP,), jnp.float32),
        pltpu.VMEM((NP,), jnp.float32),
        pltpu.VMEM((RNG,), jnp.float32),
        pltpu.VMEM_SHARED((16, NP), jnp.float32),
        pltpu.VMEM_SHARED((NP,), jnp.float32),
    ]

    @functools.partial(pl.kernel, mesh=mesh, out_type=out_type,
                       scratch_types=scratch,
                       compiler_params=pltpu.CompilerParams(
                           needs_layout_passes=False))
    def run(*refs):
        _spmm_body(with_deg, refs)

    return run


_spmm_deg = _make_spmm(True)
_spmm_nodeg = _make_spmm(False)


BM = 400


def _tc_layer_body(relu_out, x_ref, z_ref, wl, wh, wm, avl, avh, avm, attv,
                   o_ref):
    x = x_ref[...]
    z = z_ref[...]
    f32 = jnp.float32
    low = jnp.maximum(jnp.dot(z, wl[...], preferred_element_type=f32), 0.0)
    high = jnp.maximum(jnp.dot(x - z, wh[...], preferred_element_type=f32), 0.0)
    mlp = jnp.maximum(jnp.dot(x, wm[...], preferred_element_type=f32), 0.0)
    l0 = jnp.sum(low * avl[...], axis=1, keepdims=True)
    l1 = jnp.sum(high * avh[...], axis=1, keepdims=True)
    l2 = jnp.sum(mlp * avm[...], axis=1, keepdims=True)
    sig = jnp.concatenate(
        [1.0 / (1.0 + jnp.exp(-l0)),
         1.0 / (1.0 + jnp.exp(-l1)),
         1.0 / (1.0 + jnp.exp(-l2))], axis=1)
    a = attv[...]
    t = (sig[:, 0:1] * a[0:1, :] + sig[:, 1:2] * a[1:2, :]
         + sig[:, 2:3] * a[2:3, :]) / 3.0
    m = jnp.max(t, axis=1, keepdims=True)
    e = jnp.exp(t - m)
    att = e / jnp.sum(e, axis=1, keepdims=True)
    out = 3.0 * (att[:, 0:1] * low + att[:, 1:2] * high + att[:, 2:3] * mlp)
    if relu_out:
        out = jnp.maximum(out, 0.0)
    o_ref[...] = out


def _make_tc_layer(relu_out):
    blk = pl.BlockSpec((BM, C), lambda i: (i, 0))
    wspec = pl.BlockSpec((C, C), lambda i: (0, 0))
    avspec = pl.BlockSpec((1, C), lambda i: (0, 0))
    attspec = pl.BlockSpec((3, 3), lambda i: (0, 0))
    return pl.pallas_call(
        functools.partial(_tc_layer_body, relu_out),
        grid=(N // BM,),
        in_specs=[blk, blk, wspec, wspec, wspec, avspec, avspec, avspec,
                  attspec],
        out_specs=blk,
        out_shape=jax.ShapeDtypeStruct((N, C), jnp.float32),
        compiler_params=pltpu.CompilerParams(
            dimension_semantics=("parallel",)),
    )


_tc_layer1 = _make_tc_layer(True)
_tc_layer2 = _make_tc_layer(False)


def kernel(x, edge_index, w_low0, w_high0, w_mlp0, avl0, avh0, avm0, attv0,
           w_low1, w_high1, w_mlp1, avl1, avh1, avm1, attv1):
    f32 = jnp.float32
    x = x.astype(f32)
    row = edge_index[0].astype(jnp.int32)
    col = edge_index[1].astype(jnp.int32)

    xTf = x.T.reshape(-1)
    zTf1, dinv = _spmm_deg(xTf, row, col)
    z1 = zTf1.reshape(C, N).T
    fea = _tc_layer1(x, z1, w_low0, w_high0, w_mlp0,
                     avl0.T, avh0.T, avm0.T, attv0)

    feaTf = fea.T.reshape(-1)
    zTf2 = _spmm_nodeg(feaTf, row, col, dinv)
    z2 = zTf2.reshape(C, N).T
    return _tc_layer2(fea, z2, w_low1, w_high1, w_mlp1,
                      avl1.T, avh1.T, avm1.T, attv1)


# speedup vs baseline: 6.3628x; 1.2978x over previous
import functools

import jax
import jax.numpy as jnp
from jax import lax
from jax.experimental import pallas as pl
from jax.experimental.pallas import tpu as pltpu
from jax.experimental.pallas import tpu_sc as plsc

N = 10000
E = 160000
C = 256
LB = 16
F = 4
NPASS = 2
EC = 2000
NEC = E // EC
EPD = E // 16
NECD = EPD // EC
NP = 10240
RNG = NP // 16


def _zero_1d(ref, nwords, base=0):
    def z(i, _):
        ref[pl.ds(base + i * LB, LB)] = jnp.zeros((LB,), jnp.float32)
        return 0
    lax.fori_loop(0, nwords // LB, z, 0)


def _spmm_body(with_deg, refs):
    if with_deg:
        (xT, rowi, coli, zT, dinv_h,
         xb0, xb1, xb2, xb3, ac0, ac1, ac2, ac3,
         colst, rowst, degb, dinvb, tmpb, dpart, dsp, sems) = refs
    else:
        (xT, rowi, coli, dinv_i, zT,
         xb0, xb1, xb2, xb3, ac0, ac1, ac2, ac3,
         colst, rowst, degb, dinvb, tmpb, dpart, dsp, sems) = refs
    xbs = [xb0, xb1, xb2, xb3]
    accbs = [ac0, ac1, ac2, ac3]

    c = lax.axis_index("c")
    s = lax.axis_index("s")
    tid = c * 16 + s
    ones = jnp.ones((LB,), jnp.float32)

    if with_deg:
        _zero_1d(degb, NP)

        def dchunk(j, _):
            e0 = s * EPD + j * EC
            pltpu.sync_copy(coli.at[pl.ds(e0, EC)], colst.at[pl.ds(0, EC)])

            def dgrp(g, _):
                colv = colst[pl.ds(g * LB, LB)]
                plsc.addupdate_scatter(degb, [colv], ones)
                return 0
            return lax.fori_loop(0, EC // LB, dgrp, 0)
        lax.fori_loop(0, NECD, dchunk, 0)

        pltpu.sync_copy(degb, dpart.at[s])
        plsc.subcore_barrier()

        r0 = s * RNG
        _zero_1d(degb, RNG, base=r0)
        for t2 in range(16):
            pltpu.sync_copy(dpart.at[t2, pl.ds(r0, RNG)], tmpb)

            def radd(i, _):
                o = r0 + i * LB
                degb[pl.ds(o, LB)] = degb[pl.ds(o, LB)] + tmpb[pl.ds(i * LB, LB)]
                return 0
            lax.fori_loop(0, RNG // LB, radd, 0)

        def rinv(i, _):
            o = r0 + i * LB
            degb[pl.ds(o, LB)] = 1.0 / (degb[pl.ds(o, LB)] + 1.0)
            return 0
        lax.fori_loop(0, RNG // LB, rinv, 0)

        pltpu.sync_copy(degb.at[pl.ds(r0, RNG)], dsp.at[pl.ds(r0, RNG)])

        @pl.when(c == 0)
        def _():
            pltpu.sync_copy(degb.at[pl.ds(r0, RNG)], dinv_h.at[pl.ds(r0, RNG)])
        plsc.subcore_barrier()
        pltpu.sync_copy(dsp, dinvb)
    else:
        pltpu.sync_copy(dinv_i, dinvb)

    def start_stage(j, b):
        pltpu.make_async_copy(coli.at[pl.ds(j * EC, EC)],
                              colst.at[pl.ds(b * EC, EC)], sems.at[b]).start()
        pltpu.make_async_copy(rowi.at[pl.ds(j * EC, EC)],
                              rowst.at[pl.ds(b * EC, EC)], sems.at[b]).start()

    def wait_stage(b):
        pltpu.make_async_copy(coli.at[pl.ds(0, EC)],
                              colst.at[pl.ds(b * EC, EC)], sems.at[b]).wait()
        pltpu.make_async_copy(rowi.at[pl.ds(0, EC)],
                              rowst.at[pl.ds(b * EC, EC)], sems.at[b]).wait()

    for p in range(NPASS):
        fb = (tid * F + p * F * 32) * N
        for f in range(F):
            pltpu.sync_copy(xT.at[pl.ds(fb + f * N, N)], xbs[f])
            _zero_1d(accbs[f], N)

        start_stage(0, 0)

        def echunk(j, _):
            b = lax.rem(j, 2)

            @pl.when(j + 1 < NEC)
            def _():
                start_stage(j + 1, 1 - b)
            wait_stage(b)

            def egrp(g, _):
                colv = colst[pl.ds(b * EC + g * LB, LB)]
                rowv = rowst[pl.ds(b * EC + g * LB, LB)]
                for f in range(F):
                    vals = plsc.load_gather(xbs[f], [colv])
                    plsc.addupdate_scatter(accbs[f], [rowv], vals)
                return 0
            return lax.fori_loop(0, EC // LB, egrp, 0, unroll=2)
        lax.fori_loop(0, NEC, echunk, 0)

        for f in range(F):
            def fin(i, _):
                a = accbs[f][pl.ds(i * LB, LB)]
                xv = xbs[f][pl.ds(i * LB, LB)]
                dv = dinvb[pl.ds(i * LB, LB)]
                accbs[f][pl.ds(i * LB, LB)] = (a + xv) * dv
                return 0
            lax.fori_loop(0, N // LB, fin, 0)
            pltpu.sync_copy(accbs[f], zT.at[pl.ds(fb + f * N, N)])


def _make_spmm(with_deg):
    mesh = plsc.VectorSubcoreMesh(core_axis_name="c", subcore_axis_name="s")
    zT_t = jax.ShapeDtypeStruct((C * N,), jnp.float32)
    dinv_t = jax.ShapeDtypeStruct((NP,), jnp.float32)
    out_type = (zT_t, dinv_t) if with_deg else zT_t
    scratch = (
        [pltpu.VMEM((N,), jnp.float32)] * 4
        + [pltpu.VMEM((N,), jnp.float32)] * 4
        + [
            pltpu.VMEM((2 * EC,), jnp.int32),
            pltpu.VMEM((2 * EC,), jnp.int32),
            pltpu.VMEM((NP,), jnp.float32),
            pltpu.VMEM((NP,), jnp.float32),
            pltpu.VMEM((RNG,), jnp.float32),
            pltpu.VMEM_SHARED((16, NP), jnp.float32),
            pltpu.VMEM_SHARED((NP,), jnp.float32),
            pltpu.SemaphoreType.DMA((2,)),
        ])

    @functools.partial(pl.kernel, mesh=mesh, out_type=out_type,
                       scratch_types=scratch,
                       compiler_params=pltpu.CompilerParams(
                           needs_layout_passes=False))
    def run(*refs):
        _spmm_body(with_deg, refs)

    return run


_spmm_deg = _make_spmm(True)
_spmm_nodeg = _make_spmm(False)


BM = 400


def _tc_layer_body(relu_out, x_ref, z_ref, wl, wh, wm, avl, avh, avm, attv,
                   o_ref):
    x = x_ref[...]
    z = z_ref[...]
    f32 = jnp.float32
    low = jnp.maximum(jnp.dot(z, wl[...], preferred_element_type=f32), 0.0)
    high = jnp.maximum(jnp.dot(x - z, wh[...], preferred_element_type=f32), 0.0)
    mlp = jnp.maximum(jnp.dot(x, wm[...], preferred_element_type=f32), 0.0)
    l0 = jnp.sum(low * avl[...], axis=1, keepdims=True)
    l1 = jnp.sum(high * avh[...], axis=1, keepdims=True)
    l2 = jnp.sum(mlp * avm[...], axis=1, keepdims=True)
    sig = jnp.concatenate(
        [1.0 / (1.0 + jnp.exp(-l0)),
         1.0 / (1.0 + jnp.exp(-l1)),
         1.0 / (1.0 + jnp.exp(-l2))], axis=1)
    a = attv[...]
    t = (sig[:, 0:1] * a[0:1, :] + sig[:, 1:2] * a[1:2, :]
         + sig[:, 2:3] * a[2:3, :]) / 3.0
    m = jnp.max(t, axis=1, keepdims=True)
    e = jnp.exp(t - m)
    att = e / jnp.sum(e, axis=1, keepdims=True)
    out = 3.0 * (att[:, 0:1] * low + att[:, 1:2] * high + att[:, 2:3] * mlp)
    if relu_out:
        out = jnp.maximum(out, 0.0)
    o_ref[...] = out


def _make_tc_layer(relu_out):
    blk = pl.BlockSpec((BM, C), lambda i: (i, 0))
    wspec = pl.BlockSpec((C, C), lambda i: (0, 0))
    avspec = pl.BlockSpec((1, C), lambda i: (0, 0))
    attspec = pl.BlockSpec((3, 3), lambda i: (0, 0))
    return pl.pallas_call(
        functools.partial(_tc_layer_body, relu_out),
        grid=(N // BM,),
        in_specs=[blk, blk, wspec, wspec, wspec, avspec, avspec, avspec,
                  attspec],
        out_specs=blk,
        out_shape=jax.ShapeDtypeStruct((N, C), jnp.float32),
        compiler_params=pltpu.CompilerParams(
            dimension_semantics=("parallel",)),
    )


_tc_layer1 = _make_tc_layer(True)
_tc_layer2 = _make_tc_layer(False)


def kernel(x, edge_index, w_low0, w_high0, w_mlp0, avl0, avh0, avm0, attv0,
           w_low1, w_high1, w_mlp1, avl1, avh1, avm1, attv1):
    f32 = jnp.float32
    x = x.astype(f32)
    row = edge_index[0].astype(jnp.int32)
    col = edge_index[1].astype(jnp.int32)

    xTf = x.T.reshape(-1)
    zTf1, dinv = _spmm_deg(xTf, row, col)
    z1 = zTf1.reshape(C, N).T
    fea = _tc_layer1(x, z1, w_low0, w_high0, w_mlp0,
                     avl0.T, avh0.T, avm0.T, attv0)

    feaTf = fea.T.reshape(-1)
    zTf2 = _spmm_nodeg(feaTf, row, col, dinv)
    z2 = zTf2.reshape(C, N).T
    return _tc_layer2(fea, z2, w_low1, w_high1, w_mlp1,
                      avl1.T, avh1.T, avm1.T, attv1)
